# Initial kernel scaffold; baseline (speedup 1.0000x reference)
import jax
import jax.numpy as jnp
from jax.experimental import pallas as pl


def kernel(x, tables):
    raise NotImplementedError("write your pallas kernel here")


# speedup vs baseline: 1.8717x; 1.8717x over previous
import functools

import jax
import jax.numpy as jnp
from jax import lax
from jax.experimental import pallas as pl
from jax.experimental.pallas import tpu as pltpu
from jax.experimental.pallas import tpu_sc as plsc

NUM_LEVELS = 16
LEVEL_DIM = 2
BASE_RES = 16
HASHMAP_SIZE = 2 ** 19
N_POINTS = 524288
MASK = HASHMAP_SIZE - 1
_P = (1546061, 1005013, 1673733)
_CORNER_OFF = tuple(
    ((k >> 2) & 1) * _P[0] + ((k >> 1) & 1) * _P[1] + (k & 1) * _P[2]
    for k in range(8)
)

_NC = 2
_NS = 16
_NW = _NC * _NS
_C = 128
_G = _C // 16
_PW = N_POINTS // _NW
_N_ITERS = (_PW // _C) * NUM_LEVELS
_WIDE = 8
_N_WROWS = NUM_LEVELS * HASHMAP_SIZE * LEVEL_DIM // _WIDE


def _build_sc_call():
    mesh = plsc.VectorSubcoreMesh(
        core_axis_name="c", subcore_axis_name="s",
        num_cores=_NC, num_subcores=_NS,
    )

    @functools.partial(
        pl.kernel,
        mesh=mesh,
        compiler_params=pltpu.CompilerParams(
            needs_layout_passes=False, use_tc_tiling_on_sc=False),
        out_type=jax.ShapeDtypeStruct((N_POINTS, NUM_LEVELS * LEVEL_DIM),
                                      jnp.float32),
        scratch_types=[
            pltpu.VMEM((_C, 3), jnp.float32),
            pltpu.VMEM((2, 3, _C), jnp.float32),
            pltpu.VMEM((2, 8, _C), jnp.int32),
            pltpu.VMEM((2, 8, _C), jnp.int32),
            pltpu.VMEM((2, 8, _C, _WIDE), jnp.float32),
            pltpu.VMEM((_C, NUM_LEVELS * LEVEL_DIM), jnp.float32),
            pltpu.SemaphoreType.DMA,
            pltpu.SemaphoreType.DMA,
        ],
    )
    def sc_encode(x_hbm, tbl_hbm, out_hbm, xbuf, fbuf, idxbuf, lobuf, rows,
                  outbuf, sem0, sem1):
        wid = lax.axis_index("s") * _NC + lax.axis_index("c")
        iota = lax.iota(jnp.int32, 16)
        lanes = [iota + (g * 16) for g in range(_G)]
        zero16 = jnp.zeros((16,), jnp.int32)
        one16 = jnp.full((16,), 1, jnp.int32)
        two16 = jnp.full((16,), 2, jnp.int32)
        sems = (sem0, sem1)

        def split(it):
            nl = jnp.int32(NUM_LEVELS)
            return lax.div(it, nl), lax.rem(it, nl)

        def pass1(it, p):
            chunk, lvl = split(it)
            base = wid * _PW + chunk * _C

            @pl.when(lvl == 0)
            def _():
                pltpu.sync_copy(x_hbm.at[pl.ds(base, _C)], xbuf)

            lvl_vec = jnp.full((16,), lvl, jnp.int32)
            res_vec = jnp.left_shift(
                jnp.full((16,), BASE_RES, jnp.int32), lvl_vec
            ).astype(jnp.float32)
            lvl_mix = lvl_vec + jnp.left_shift(lvl_vec, 19)
            for g in range(_G):
                lane = lanes[g]
                fi = []
                for d, dvec in ((0, zero16), (1, one16), (2, two16)):
                    xd = plsc.load_gather(xbuf, [lane, dvec])
                    xn = xd * 0.5 + 0.5
                    xn = jnp.minimum(jnp.maximum(xn, 0.0), 1.0 - 1e-6)
                    sc = xn * res_vec
                    fid = sc.astype(jnp.int32)
                    fi.append(fid)
                    fbuf[p, d, pl.ds(g * 16, 16)] = (
                        sc - fid.astype(jnp.float32))
                hb = fi[0] * _P[0] + fi[1] * _P[1] + fi[2] * _P[2]
                for k in range(8):
                    hk = hb + _CORNER_OFF[k] if _CORNER_OFF[k] else hb
                    e = (hk & MASK) ^ lvl_mix
                    idxbuf[p, k, pl.ds(g * 16, 16)] = lax.shift_right_logical(
                        e, jnp.int32(2))
                    lobuf[p, k, pl.ds(g * 16, 16)] = jnp.left_shift(
                        e & 3, jnp.int32(1))

        def fire(p):
            for k in range(8):
                pltpu.async_copy(
                    tbl_hbm.at[idxbuf.at[jnp.int32(p)].at[jnp.int32(k)]],
                    rows.at[jnp.int32(p)].at[jnp.int32(k)], sems[p])

        def drain(p):
            for k in range(8):
                pltpu.make_async_copy(
                    tbl_hbm.at[idxbuf.at[jnp.int32(p)].at[jnp.int32(k)]],
                    rows.at[jnp.int32(p)].at[jnp.int32(k)], sems[p]).wait()

        def pass2(it, p):
            chunk, lvl = split(it)
            base = wid * _PW + chunk * _C
            lvl_vec = jnp.full((16,), lvl, jnp.int32)
            col0 = lvl_vec + lvl_vec
            col1 = col0 + 1
            for g in range(_G):
                lane = lanes[g]
                f0 = fbuf[p, 0, pl.ds(g * 16, 16)]
                f1 = fbuf[p, 1, pl.ds(g * 16, 16)]
                f2 = fbuf[p, 2, pl.ds(g * 16, 16)]
                s0 = 1.0 - f0
                s1 = 1.0 - f1
                s2 = 1.0 - f2
                p00 = s0 * s1
                p01 = s0 * f1
                p10 = f0 * s1
                p11 = f0 * f1
                w = (p00 * s2, p00 * f2, p01 * s2, p01 * f2,
                     p10 * s2, p10 * f2, p11 * s2, p11 * f2)
                acc0 = jnp.zeros((16,), jnp.float32)
                acc1 = jnp.zeros((16,), jnp.float32)
                for k in range(8):
                    k16 = jnp.full((16,), k, jnp.int32)
                    sub0 = lobuf[p, k, pl.ds(g * 16, 16)]
                    r0 = plsc.load_gather(rows.at[jnp.int32(p)],
                                          [k16, lane, sub0])
                    r1 = plsc.load_gather(rows.at[jnp.int32(p)],
                                          [k16, lane, sub0 + 1])
                    acc0 = acc0 + w[k] * r0
                    acc1 = acc1 + w[k] * r1
                plsc.store_scatter(outbuf, [lane, col0], acc0)
                plsc.store_scatter(outbuf, [lane, col1], acc1)

            @pl.when(lvl == NUM_LEVELS - 1)
            def _():
                pltpu.sync_copy(outbuf, out_hbm.at[pl.ds(base, _C)])

        @pl.loop(jnp.int32(0), jnp.int32(_N_ITERS // 2))
        def _loop(j):
            j = j.astype(jnp.int32)
            it0 = j * 2
            it1 = it0 + 1
            pass1(it0, 0)
            fire(0)

            @pl.when(j > 0)
            def _():
                drain(1)
                pass2(it0 - 1, 1)

            pass1(it1, 1)
            fire(1)
            drain(0)
            pass2(it0, 0)

        drain(1)
        pass2(jnp.int32(_N_ITERS - 1), 1)

    return sc_encode


@functools.lru_cache(maxsize=None)
def _get_sc_call():
    return _build_sc_call()


def kernel(x, tables):
    twide = tables.reshape(_N_WROWS, _WIDE)
    return _get_sc_call()(x, twide)


# speedup vs baseline: 6.0463x; 3.2304x over previous
import functools

import jax
import jax.numpy as jnp
from jax import lax
from jax.experimental import pallas as pl
from jax.experimental.pallas import tpu as pltpu
from jax.experimental.pallas import tpu_sc as plsc

NUM_LEVELS = 16
LEVEL_DIM = 2
BASE_RES = 16
HASHMAP_SIZE = 2 ** 19
N_POINTS = 524288
MASK = HASHMAP_SIZE - 1
_P = (1546061, 1005013, 1673733)
_CORNER_OFF = tuple(
    ((k >> 2) & 1) * _P[0] + ((k >> 1) & 1) * _P[1] + (k & 1) * _P[2]
    for k in range(8)
)

_NC = 2
_NS = 16
_NW = _NC * _NS
_C = 128
_G = _C // 16
_PW = N_POINTS // _NW
_N_ITERS = (_PW // _C) * NUM_LEVELS
_WIDE = 8
_N_WROWS = NUM_LEVELS * HASHMAP_SIZE * LEVEL_DIM // _WIDE

_UNITS = NUM_LEVELS * HASHMAP_SIZE // 128
_U_PER_W = _UNITS // _NW
_U_BATCH = 16
_A_ITERS = _U_PER_W // _U_BATCH


def _build_linearize():
    mesh = plsc.VectorSubcoreMesh(
        core_axis_name="c", subcore_axis_name="s",
        num_cores=_NC, num_subcores=_NS,
    )

    @functools.partial(
        pl.kernel,
        mesh=mesh,
        compiler_params=pltpu.CompilerParams(
            needs_layout_passes=False, use_tc_tiling_on_sc=False),
        out_type=jax.ShapeDtypeStruct((_N_WROWS, _WIDE), jnp.float32),
        scratch_types=[
            pltpu.VMEM((_U_BATCH * 32, 8), jnp.float32),
            pltpu.VMEM((_U_BATCH * 32, 8), jnp.float32),
        ],
    )
    def sc_linearize(raw_hbm, tlin_hbm, abuf_in, abuf_out):
        wid = lax.axis_index("s") * _NC + lax.axis_index("c")
        iota = lax.iota(jnp.int32, 16)
        srow, scol, drow, dcol = [], [], [], []
        for g in range(16):
            t = iota + g * 16
            src = (t & 1) * 128 + lax.shift_right_logical(t, jnp.int32(1))
            srow.append(lax.shift_right_logical(src, jnp.int32(3)))
            scol.append(src & 7)
            drow.append(lax.shift_right_logical(t, jnp.int32(3)))
            dcol.append(t & 7)

        @pl.loop(jnp.int32(0), jnp.int32(_A_ITERS))
        def _alin(i):
            i = i.astype(jnp.int32)
            r0 = (wid * _U_PER_W + i * _U_BATCH) * 32
            pltpu.sync_copy(raw_hbm.at[pl.ds(r0, _U_BATCH * 32)], abuf_in)
            for ul in range(_U_BATCH):
                off = ul * 32
                for g in range(16):
                    v = plsc.load_gather(abuf_in, [srow[g] + off, scol[g]])
                    plsc.store_scatter(abuf_out, [drow[g] + off, dcol[g]], v)
            pltpu.sync_copy(abuf_out, tlin_hbm.at[pl.ds(r0, _U_BATCH * 32)])

    return sc_linearize


def _build_encode():
    mesh = plsc.VectorSubcoreMesh(
        core_axis_name="c", subcore_axis_name="s",
        num_cores=_NC, num_subcores=_NS,
    )

    @functools.partial(
        pl.kernel,
        mesh=mesh,
        compiler_params=pltpu.CompilerParams(
            needs_layout_passes=False, use_tc_tiling_on_sc=False),
        out_type=jax.ShapeDtypeStruct((N_POINTS, NUM_LEVELS * LEVEL_DIM),
                                      jnp.float32),
        scratch_types=[
            pltpu.VMEM((_C, 3), jnp.float32),
            pltpu.VMEM((2, 3, _C), jnp.float32),
            pltpu.VMEM((2, 8, _C), jnp.int32),
            pltpu.VMEM((2, 8, _C), jnp.int32),
            pltpu.VMEM((2, 8, _C, _WIDE), jnp.float32),
            pltpu.VMEM((_C, NUM_LEVELS * LEVEL_DIM), jnp.float32),
            pltpu.SemaphoreType.DMA,
            pltpu.SemaphoreType.DMA,
        ],
    )
    def sc_encode(x_hbm, tbl_hbm, out_hbm, xbuf, fbuf, idxbuf, lobuf, rows,
                  outbuf, sem0, sem1):
        wid = lax.axis_index("s") * _NC + lax.axis_index("c")
        iota = lax.iota(jnp.int32, 16)
        lanes = [iota + (g * 16) for g in range(_G)]
        zero16 = jnp.zeros((16,), jnp.int32)
        one16 = jnp.full((16,), 1, jnp.int32)
        two16 = jnp.full((16,), 2, jnp.int32)
        sems = (sem0, sem1)

        def split(it):
            nl = jnp.int32(NUM_LEVELS)
            return lax.div(it, nl), lax.rem(it, nl)

        def pass1(it, p):
            chunk, lvl = split(it)
            base = wid * _PW + chunk * _C

            @pl.when(lvl == 0)
            def _():
                pltpu.sync_copy(x_hbm.at[pl.ds(base, _C)], xbuf)

            lvl_vec = jnp.full((16,), lvl, jnp.int32)
            res_vec = jnp.left_shift(
                jnp.full((16,), BASE_RES, jnp.int32), lvl_vec
            ).astype(jnp.float32)
            lvl_mix = lvl_vec + jnp.left_shift(lvl_vec, 19)
            for g in range(_G):
                lane = lanes[g]
                fi = []
                for d, dvec in ((0, zero16), (1, one16), (2, two16)):
                    xd = plsc.load_gather(xbuf, [lane, dvec])
                    xn = xd * 0.5 + 0.5
                    xn = jnp.minimum(jnp.maximum(xn, 0.0), 1.0 - 1e-6)
                    sc = xn * res_vec
                    fid = sc.astype(jnp.int32)
                    fi.append(fid)
                    fbuf[p, d, pl.ds(g * 16, 16)] = (
                        sc - fid.astype(jnp.float32))
                hb = fi[0] * _P[0] + fi[1] * _P[1] + fi[2] * _P[2]
                for k in range(8):
                    hk = hb + _CORNER_OFF[k] if _CORNER_OFF[k] else hb
                    e = (hk & MASK) ^ lvl_mix
                    idxbuf[p, k, pl.ds(g * 16, 16)] = lax.shift_right_logical(
                        e, jnp.int32(2))
                    lobuf[p, k, pl.ds(g * 16, 16)] = jnp.left_shift(
                        e & 3, jnp.int32(1))

        def fire(p):
            for k in range(8):
                pltpu.async_copy(
                    tbl_hbm.at[idxbuf.at[jnp.int32(p)].at[jnp.int32(k)]],
                    rows.at[jnp.int32(p)].at[jnp.int32(k)], sems[p])

        def drain(p):
            for k in range(8):
                pltpu.make_async_copy(
                    tbl_hbm.at[idxbuf.at[jnp.int32(p)].at[jnp.int32(k)]],
                    rows.at[jnp.int32(p)].at[jnp.int32(k)], sems[p]).wait()

        def pass2(it, p):
            chunk, lvl = split(it)
            base = wid * _PW + chunk * _C
            lvl_vec = jnp.full((16,), lvl, jnp.int32)
            col0 = lvl_vec + lvl_vec
            col1 = col0 + 1
            for g in range(_G):
                lane = lanes[g]
                f0 = fbuf[p, 0, pl.ds(g * 16, 16)]
                f1 = fbuf[p, 1, pl.ds(g * 16, 16)]
                f2 = fbuf[p, 2, pl.ds(g * 16, 16)]
                s0 = 1.0 - f0
                s1 = 1.0 - f1
                s2 = 1.0 - f2
                p00 = s0 * s1
                p01 = s0 * f1
                p10 = f0 * s1
                p11 = f0 * f1
                w = (p00 * s2, p00 * f2, p01 * s2, p01 * f2,
                     p10 * s2, p10 * f2, p11 * s2, p11 * f2)
                acc0 = jnp.zeros((16,), jnp.float32)
                acc1 = jnp.zeros((16,), jnp.float32)
                for k in range(8):
                    k16 = jnp.full((16,), k, jnp.int32)
                    sub0 = lobuf[p, k, pl.ds(g * 16, 16)]
                    r0 = plsc.load_gather(rows.at[jnp.int32(p)],
                                          [k16, lane, sub0])
                    r1 = plsc.load_gather(rows.at[jnp.int32(p)],
                                          [k16, lane, sub0 + 1])
                    acc0 = acc0 + w[k] * r0
                    acc1 = acc1 + w[k] * r1
                plsc.store_scatter(outbuf, [lane, col0], acc0)
                plsc.store_scatter(outbuf, [lane, col1], acc1)

            @pl.when(lvl == NUM_LEVELS - 1)
            def _():
                pltpu.sync_copy(outbuf, out_hbm.at[pl.ds(base, _C)])

        @pl.loop(jnp.int32(0), jnp.int32(_N_ITERS // 2))
        def _loop(j):
            j = j.astype(jnp.int32)
            it0 = j * 2
            it1 = it0 + 1
            pass1(it0, 0)
            fire(0)

            @pl.when(j > 0)
            def _():
                drain(1)
                pass2(it0 - 1, 1)

            pass1(it1, 1)
            fire(1)
            drain(0)
            pass2(it0, 0)

        drain(1)
        pass2(jnp.int32(_N_ITERS - 1), 1)

    return sc_encode


@functools.lru_cache(maxsize=None)
def _get_calls():
    return _build_linearize(), _build_encode()


def kernel(x, tables):
    braw = tables.reshape(NUM_LEVELS, HASHMAP_SIZE // 128, 128, LEVEL_DIM)
    braw = braw.transpose(0, 1, 3, 2).reshape(_N_WROWS, _WIDE)
    linearize, encode = _get_calls()
    tlin = linearize(braw)
    return encode(x, tlin)


# speedup vs baseline: 6.3997x; 1.0584x over previous
import functools

import jax
import jax.numpy as jnp
from jax import lax
from jax.experimental import pallas as pl
from jax.experimental.pallas import tpu as pltpu
from jax.experimental.pallas import tpu_sc as plsc

NUM_LEVELS = 16
LEVEL_DIM = 2
BASE_RES = 16
HASHMAP_SIZE = 2 ** 19
N_POINTS = 524288
MASK = HASHMAP_SIZE - 1
_P = (1546061, 1005013, 1673733)
_CORNER_OFF = tuple(
    ((k >> 2) & 1) * _P[0] + ((k >> 1) & 1) * _P[1] + (k & 1) * _P[2]
    for k in range(8)
)

_NC = 2
_NS = 16
_NW = _NC * _NS
_C = 128
_G = _C // 16
_PW = N_POINTS // _NW
_N_ITERS = (_PW // _C) * NUM_LEVELS
_WIDE = 8
_N_WROWS = NUM_LEVELS * HASHMAP_SIZE * LEVEL_DIM // _WIDE

_UNITS = NUM_LEVELS * HASHMAP_SIZE // 128
_U_PER_W = _UNITS // _NW
_U_BATCH = 16
_A_ITERS = _U_PER_W // _U_BATCH


def _build_linearize():
    mesh = plsc.VectorSubcoreMesh(
        core_axis_name="c", subcore_axis_name="s",
        num_cores=_NC, num_subcores=_NS,
    )

    @functools.partial(
        pl.kernel,
        mesh=mesh,
        compiler_params=pltpu.CompilerParams(
            needs_layout_passes=False, use_tc_tiling_on_sc=False),
        out_type=jax.ShapeDtypeStruct((_N_WROWS, _WIDE), jnp.float32),
        scratch_types=[
            pltpu.VMEM((_U_BATCH * 32, 8), jnp.float32),
            pltpu.VMEM((_U_BATCH * 32, 8), jnp.float32),
        ],
    )
    def sc_linearize(raw_hbm, tlin_hbm, abuf_in, abuf_out):
        wid = lax.axis_index("s") * _NC + lax.axis_index("c")
        iota = lax.iota(jnp.int32, 16)
        srow, scol, drow, dcol = [], [], [], []
        for g in range(16):
            t = iota + g * 16
            src = (t & 1) * 128 + lax.shift_right_logical(t, jnp.int32(1))
            srow.append(lax.shift_right_logical(src, jnp.int32(3)))
            scol.append(src & 7)
            drow.append(lax.shift_right_logical(t, jnp.int32(3)))
            dcol.append(t & 7)

        @pl.loop(jnp.int32(0), jnp.int32(_A_ITERS))
        def _alin(i):
            i = i.astype(jnp.int32)
            r0 = (wid * _U_PER_W + i * _U_BATCH) * 32
            pltpu.sync_copy(raw_hbm.at[pl.ds(r0, _U_BATCH * 32)], abuf_in)
            for ul in range(_U_BATCH):
                off = ul * 32
                for g in range(16):
                    v = plsc.load_gather(abuf_in, [srow[g] + off, scol[g]])
                    plsc.store_scatter(abuf_out, [drow[g] + off, dcol[g]], v)
            pltpu.sync_copy(abuf_out, tlin_hbm.at[pl.ds(r0, _U_BATCH * 32)])

    return sc_linearize


def _build_encode():
    mesh = plsc.VectorSubcoreMesh(
        core_axis_name="c", subcore_axis_name="s",
        num_cores=_NC, num_subcores=_NS,
    )

    @functools.partial(
        pl.kernel,
        mesh=mesh,
        compiler_params=pltpu.CompilerParams(
            needs_layout_passes=False, use_tc_tiling_on_sc=False),
        out_type=jax.ShapeDtypeStruct((N_POINTS // 4, 128), jnp.float32),
        scratch_types=[
            pltpu.VMEM((_C, 3), jnp.float32),
            pltpu.VMEM((2, 3, _C), jnp.float32),
            pltpu.VMEM((2, 8, _C), jnp.int32),
            pltpu.VMEM((2, 8, _C), jnp.int32),
            pltpu.VMEM((2, 8, _C, _WIDE), jnp.float32),
            pltpu.VMEM((4, 8, _C), jnp.float32),
            pltpu.SemaphoreType.DMA,
            pltpu.SemaphoreType.DMA,
        ],
    )
    def sc_encode(x_hbm, tbl_hbm, out_hbm, xbuf, fbuf, idxbuf, lobuf, rows,
                  outbuf, sem0, sem1):
        wid = lax.axis_index("s") * _NC + lax.axis_index("c")
        iota = lax.iota(jnp.int32, 16)
        lanes = [iota + (g * 16) for g in range(_G)]
        zero16 = jnp.zeros((16,), jnp.int32)
        one16 = jnp.full((16,), 1, jnp.int32)
        two16 = jnp.full((16,), 2, jnp.int32)
        sems = (sem0, sem1)

        def split(it):
            nl = jnp.int32(NUM_LEVELS)
            return lax.div(it, nl), lax.rem(it, nl)

        def pass1(it, p):
            chunk, lvl = split(it)
            base = wid * _PW + chunk * _C

            @pl.when(lvl == 0)
            def _():
                pltpu.sync_copy(x_hbm.at[pl.ds(base, _C)], xbuf)

            lvl_vec = jnp.full((16,), lvl, jnp.int32)
            res_vec = jnp.left_shift(
                jnp.full((16,), BASE_RES, jnp.int32), lvl_vec
            ).astype(jnp.float32)
            lvl_mix = lvl_vec + jnp.left_shift(lvl_vec, 19)
            for g in range(_G):
                lane = lanes[g]
                fi = []
                for d, dvec in ((0, zero16), (1, one16), (2, two16)):
                    xd = plsc.load_gather(xbuf, [lane, dvec])
                    xn = xd * 0.5 + 0.5
                    xn = jnp.minimum(jnp.maximum(xn, 0.0), 1.0 - 1e-6)
                    sc = xn * res_vec
                    fid = sc.astype(jnp.int32)
                    fi.append(fid)
                    fbuf[p, d, pl.ds(g * 16, 16)] = (
                        sc - fid.astype(jnp.float32))
                hb = fi[0] * _P[0] + fi[1] * _P[1] + fi[2] * _P[2]
                for k in range(8):
                    hk = hb + _CORNER_OFF[k] if _CORNER_OFF[k] else hb
                    e = (hk & MASK) ^ lvl_mix
                    idxbuf[p, k, pl.ds(g * 16, 16)] = lax.shift_right_logical(
                        e, jnp.int32(2))
                    lobuf[p, k, pl.ds(g * 16, 16)] = jnp.left_shift(
                        e & 3, jnp.int32(1))

        def fire(p):
            for k in range(8):
                pltpu.async_copy(
                    tbl_hbm.at[idxbuf.at[jnp.int32(p)].at[jnp.int32(k)]],
                    rows.at[jnp.int32(p)].at[jnp.int32(k)], sems[p])

        def drain(p):
            for k in range(8):
                pltpu.make_async_copy(
                    tbl_hbm.at[idxbuf.at[jnp.int32(p)].at[jnp.int32(k)]],
                    rows.at[jnp.int32(p)].at[jnp.int32(k)], sems[p]).wait()

        def pass2(it, p):
            chunk, lvl = split(it)
            lvl_vec = jnp.full((16,), lvl, jnp.int32)
            cb_vec = lax.shift_right_logical(lvl_vec, jnp.int32(2))
            cr0 = (lvl_vec & 3) + (lvl_vec & 3)
            cr1 = cr0 + 1
            for g in range(_G):
                lane = lanes[g]
                f0 = fbuf[p, 0, pl.ds(g * 16, 16)]
                f1 = fbuf[p, 1, pl.ds(g * 16, 16)]
                f2 = fbuf[p, 2, pl.ds(g * 16, 16)]
                s0 = 1.0 - f0
                s1 = 1.0 - f1
                s2 = 1.0 - f2
                p00 = s0 * s1
                p01 = s0 * f1
                p10 = f0 * s1
                p11 = f0 * f1
                w = (p00 * s2, p00 * f2, p01 * s2, p01 * f2,
                     p10 * s2, p10 * f2, p11 * s2, p11 * f2)
                acc0 = jnp.zeros((16,), jnp.float32)
                acc1 = jnp.zeros((16,), jnp.float32)
                for k in range(8):
                    k16 = jnp.full((16,), k, jnp.int32)
                    sub0 = lobuf[p, k, pl.ds(g * 16, 16)]
                    r0 = plsc.load_gather(rows.at[jnp.int32(p)],
                                          [k16, lane, sub0])
                    r1 = plsc.load_gather(rows.at[jnp.int32(p)],
                                          [k16, lane, sub0 + 1])
                    acc0 = acc0 + w[k] * r0
                    acc1 = acc1 + w[k] * r1
                plsc.store_scatter(outbuf, [cb_vec, cr0, lane], acc0)
                plsc.store_scatter(outbuf, [cb_vec, cr1, lane], acc1)

            @pl.when(lvl == NUM_LEVELS - 1)
            def _():
                pb = wid * (_PW // _C) + chunk
                for cb in range(4):
                    pltpu.sync_copy(
                        outbuf.at[jnp.int32(cb)],
                        out_hbm.at[pl.ds((cb * 4096 + pb) * 8, 8)])

        @pl.loop(jnp.int32(0), jnp.int32(_N_ITERS // 2))
        def _loop(j):
            j = j.astype(jnp.int32)
            it0 = j * 2
            it1 = it0 + 1
            pass1(it0, 0)
            fire(0)

            @pl.when(j > 0)
            def _():
                drain(1)
                pass2(it0 - 1, 1)

            pass1(it1, 1)
            fire(1)
            drain(0)
            pass2(it0, 0)

        drain(1)
        pass2(jnp.int32(_N_ITERS - 1), 1)

    return sc_encode


@functools.lru_cache(maxsize=None)
def _get_calls():
    return _build_linearize(), _build_encode()


def kernel(x, tables):
    braw = tables.reshape(NUM_LEVELS, HASHMAP_SIZE // 128, 128, LEVEL_DIM)
    braw = braw.transpose(0, 1, 3, 2).reshape(_N_WROWS, _WIDE)
    linearize, encode = _get_calls()
    tlin = linearize(braw)
    out_raw = encode(x, tlin)
    out = out_raw.reshape(4, N_POINTS // 128, 8, 128)
    return out.transpose(1, 3, 0, 2).reshape(N_POINTS, NUM_LEVELS * LEVEL_DIM)


# speedup vs baseline: 6.5437x; 1.0225x over previous
import functools

import jax
import jax.numpy as jnp
from jax import lax
from jax.experimental import pallas as pl
from jax.experimental.pallas import tpu as pltpu
from jax.experimental.pallas import tpu_sc as plsc

NUM_LEVELS = 16
LEVEL_DIM = 2
BASE_RES = 16
HASHMAP_SIZE = 2 ** 19
N_POINTS = 524288
MASK = HASHMAP_SIZE - 1
_P = (1546061, 1005013, 1673733)
_CORNER_OFF = tuple(
    ((k >> 2) & 1) * _P[0] + ((k >> 1) & 1) * _P[1] + (k & 1) * _P[2]
    for k in range(8)
)

_NC = 2
_NS = 16
_NW = _NC * _NS
_C = 128
_G = _C // 16
_PW = N_POINTS // _NW
_N_ITERS = (_PW // _C) * NUM_LEVELS
_WIDE = 8
_N_WROWS = NUM_LEVELS * HASHMAP_SIZE * LEVEL_DIM // _WIDE

_UNITS = NUM_LEVELS * HASHMAP_SIZE // 128
_U_PER_W = _UNITS // _NW
_U_BATCH = 16
_A_ITERS = _U_PER_W // _U_BATCH


def _build_linearize():
    mesh = plsc.VectorSubcoreMesh(
        core_axis_name="c", subcore_axis_name="s",
        num_cores=_NC, num_subcores=_NS,
    )

    @functools.partial(
        pl.kernel,
        mesh=mesh,
        compiler_params=pltpu.CompilerParams(
            needs_layout_passes=False, use_tc_tiling_on_sc=False),
        out_type=jax.ShapeDtypeStruct((_N_WROWS, _WIDE), jnp.float32),
        scratch_types=[
            pltpu.VMEM((2, _U_BATCH * 32, 8), jnp.float32),
            pltpu.VMEM((2, _U_BATCH * 32, 8), jnp.float32),
            pltpu.SemaphoreType.DMA,
            pltpu.SemaphoreType.DMA,
            pltpu.SemaphoreType.DMA,
            pltpu.SemaphoreType.DMA,
        ],
    )
    def sc_linearize(raw_hbm, tlin_hbm, abuf_in, abuf_out,
                     isem0, isem1, osem0, osem1):
        wid = lax.axis_index("s") * _NC + lax.axis_index("c")
        iota = lax.iota(jnp.int32, 16)
        srow, scol, drow, dcol = [], [], [], []
        for g in range(16):
            t = iota + g * 16
            src = (t & 1) * 128 + lax.shift_right_logical(t, jnp.int32(1))
            srow.append(lax.shift_right_logical(src, jnp.int32(3)))
            scol.append(src & 7)
            drow.append(lax.shift_right_logical(t, jnp.int32(3)))
            dcol.append(t & 7)

        isems = (isem0, isem1)
        osems = (osem0, osem1)
        nrows = _U_BATCH * 32
        base_r = wid * _U_PER_W * 32

        def row0(i):
            return base_r + i * nrows

        def fire_in(i, p):
            pltpu.async_copy(raw_hbm.at[pl.ds(row0(i), nrows)],
                             abuf_in.at[jnp.int32(p)], isems[p])

        def wait_in(i, p):
            pltpu.make_async_copy(raw_hbm.at[pl.ds(row0(i), nrows)],
                                  abuf_in.at[jnp.int32(p)], isems[p]).wait()

        def fire_out(i, p):
            pltpu.async_copy(abuf_out.at[jnp.int32(p)],
                             tlin_hbm.at[pl.ds(row0(i), nrows)], osems[p])

        def wait_out(i, p):
            pltpu.make_async_copy(abuf_out.at[jnp.int32(p)],
                                  tlin_hbm.at[pl.ds(row0(i), nrows)],
                                  osems[p]).wait()

        def interleave(p):
            for ul in range(_U_BATCH):
                off = ul * 32
                for g in range(16):
                    v = plsc.load_gather(abuf_in.at[jnp.int32(p)],
                                         [srow[g] + off, scol[g]])
                    plsc.store_scatter(abuf_out.at[jnp.int32(p)],
                                       [drow[g] + off, dcol[g]], v)

        fire_in(jnp.int32(0), 0)
        fire_in(jnp.int32(1), 1)

        @pl.loop(jnp.int32(0), jnp.int32(_A_ITERS // 2))
        def _alin(j):
            j = j.astype(jnp.int32)
            i0 = j * 2
            i1 = i0 + 1
            wait_in(i0, 0)

            @pl.when(j > 0)
            def _():
                wait_out(i0 - 2, 0)

            interleave(0)
            fire_out(i0, 0)

            @pl.when(i0 + 2 < _A_ITERS)
            def _():
                fire_in(i0 + 2, 0)

            wait_in(i1, 1)

            @pl.when(j > 0)
            def _():
                wait_out(i1 - 2, 1)

            interleave(1)
            fire_out(i1, 1)

            @pl.when(i1 + 2 < _A_ITERS)
            def _():
                fire_in(i1 + 2, 1)

        wait_out(jnp.int32(_A_ITERS - 2), 0)
        wait_out(jnp.int32(_A_ITERS - 1), 1)

    return sc_linearize


def _build_encode():
    mesh = plsc.VectorSubcoreMesh(
        core_axis_name="c", subcore_axis_name="s",
        num_cores=_NC, num_subcores=_NS,
    )

    @functools.partial(
        pl.kernel,
        mesh=mesh,
        compiler_params=pltpu.CompilerParams(
            needs_layout_passes=False, use_tc_tiling_on_sc=False),
        out_type=jax.ShapeDtypeStruct((N_POINTS // 4, 128), jnp.float32),
        scratch_types=[
            pltpu.VMEM((_C, 3), jnp.float32),
            pltpu.VMEM((2, 3, _C), jnp.float32),
            pltpu.VMEM((2, 8, _C), jnp.int32),
            pltpu.VMEM((2, 8, _C), jnp.int32),
            pltpu.VMEM((2, 8, _C, _WIDE), jnp.float32),
            pltpu.VMEM((4, 8, _C), jnp.float32),
            pltpu.SemaphoreType.DMA,
            pltpu.SemaphoreType.DMA,
        ],
    )
    def sc_encode(x_hbm, tbl_hbm, out_hbm, xbuf, fbuf, idxbuf, lobuf, rows,
                  outbuf, sem0, sem1):
        wid = lax.axis_index("s") * _NC + lax.axis_index("c")
        iota = lax.iota(jnp.int32, 16)
        lanes = [iota + (g * 16) for g in range(_G)]
        zero16 = jnp.zeros((16,), jnp.int32)
        one16 = jnp.full((16,), 1, jnp.int32)
        two16 = jnp.full((16,), 2, jnp.int32)
        sems = (sem0, sem1)

        def split(it):
            nl = jnp.int32(NUM_LEVELS)
            return lax.div(it, nl), lax.rem(it, nl)

        def pass1(it, p):
            chunk, lvl = split(it)
            base = wid * _PW + chunk * _C

            @pl.when(lvl == 0)
            def _():
                pltpu.sync_copy(x_hbm.at[pl.ds(base, _C)], xbuf)

            lvl_vec = jnp.full((16,), lvl, jnp.int32)
            res_vec = jnp.left_shift(
                jnp.full((16,), BASE_RES, jnp.int32), lvl_vec
            ).astype(jnp.float32)
            lvl_mix = lvl_vec + jnp.left_shift(lvl_vec, 19)
            for g in range(_G):
                lane = lanes[g]
                fi = []
                for d, dvec in ((0, zero16), (1, one16), (2, two16)):
                    xd = plsc.load_gather(xbuf, [lane, dvec])
                    xn = xd * 0.5 + 0.5
                    xn = jnp.minimum(jnp.maximum(xn, 0.0), 1.0 - 1e-6)
                    sc = xn * res_vec
                    fid = sc.astype(jnp.int32)
                    fi.append(fid)
                    fbuf[p, d, pl.ds(g * 16, 16)] = (
                        sc - fid.astype(jnp.float32))
                hb = fi[0] * _P[0] + fi[1] * _P[1] + fi[2] * _P[2]
                for k in range(8):
                    hk = hb + _CORNER_OFF[k] if _CORNER_OFF[k] else hb
                    e = (hk & MASK) ^ lvl_mix
                    idxbuf[p, k, pl.ds(g * 16, 16)] = lax.shift_right_logical(
                        e, jnp.int32(2))
                    lobuf[p, k, pl.ds(g * 16, 16)] = jnp.left_shift(
                        e & 3, jnp.int32(1))

        def fire(p):
            for k in range(8):
                pltpu.async_copy(
                    tbl_hbm.at[idxbuf.at[jnp.int32(p)].at[jnp.int32(k)]],
                    rows.at[jnp.int32(p)].at[jnp.int32(k)], sems[p])

        def drain(p):
            for k in range(8):
                pltpu.make_async_copy(
                    tbl_hbm.at[idxbuf.at[jnp.int32(p)].at[jnp.int32(k)]],
                    rows.at[jnp.int32(p)].at[jnp.int32(k)], sems[p]).wait()

        def pass2(it, p):
            chunk, lvl = split(it)
            lvl_vec = jnp.full((16,), lvl, jnp.int32)
            cb_vec = lax.shift_right_logical(lvl_vec, jnp.int32(2))
            cr0 = (lvl_vec & 3) + (lvl_vec & 3)
            cr1 = cr0 + 1
            for g in range(_G):
                lane = lanes[g]
                f0 = fbuf[p, 0, pl.ds(g * 16, 16)]
                f1 = fbuf[p, 1, pl.ds(g * 16, 16)]
                f2 = fbuf[p, 2, pl.ds(g * 16, 16)]
                s0 = 1.0 - f0
                s1 = 1.0 - f1
                s2 = 1.0 - f2
                p00 = s0 * s1
                p01 = s0 * f1
                p10 = f0 * s1
                p11 = f0 * f1
                w = (p00 * s2, p00 * f2, p01 * s2, p01 * f2,
                     p10 * s2, p10 * f2, p11 * s2, p11 * f2)
                acc0 = jnp.zeros((16,), jnp.float32)
                acc1 = jnp.zeros((16,), jnp.float32)
                for k in range(8):
                    k16 = jnp.full((16,), k, jnp.int32)
                    sub0 = lobuf[p, k, pl.ds(g * 16, 16)]
                    r0 = plsc.load_gather(rows.at[jnp.int32(p)],
                                          [k16, lane, sub0])
                    r1 = plsc.load_gather(rows.at[jnp.int32(p)],
                                          [k16, lane, sub0 + 1])
                    acc0 = acc0 + w[k] * r0
                    acc1 = acc1 + w[k] * r1
                plsc.store_scatter(outbuf, [cb_vec, cr0, lane], acc0)
                plsc.store_scatter(outbuf, [cb_vec, cr1, lane], acc1)

            @pl.when(lvl == NUM_LEVELS - 1)
            def _():
                pb = wid * (_PW // _C) + chunk
                for cb in range(4):
                    pltpu.sync_copy(
                        outbuf.at[jnp.int32(cb)],
                        out_hbm.at[pl.ds((cb * 4096 + pb) * 8, 8)])

        @pl.loop(jnp.int32(0), jnp.int32(_N_ITERS // 2))
        def _loop(j):
            j = j.astype(jnp.int32)
            it0 = j * 2
            it1 = it0 + 1
            pass1(it0, 0)
            fire(0)

            @pl.when(j > 0)
            def _():
                drain(1)
                pass2(it0 - 1, 1)

            pass1(it1, 1)
            fire(1)
            drain(0)
            pass2(it0, 0)

        drain(1)
        pass2(jnp.int32(_N_ITERS - 1), 1)

    return sc_encode


@functools.lru_cache(maxsize=None)
def _get_calls():
    return _build_linearize(), _build_encode()


def kernel(x, tables):
    braw = tables.reshape(NUM_LEVELS, HASHMAP_SIZE // 128, 128, LEVEL_DIM)
    braw = braw.transpose(0, 1, 3, 2).reshape(_N_WROWS, _WIDE)
    linearize, encode = _get_calls()
    tlin = linearize(braw)
    out_raw = encode(x, tlin)
    out = out_raw.reshape(4, N_POINTS // 128, 8, 128)
    return out.transpose(1, 3, 0, 2).reshape(N_POINTS, NUM_LEVELS * LEVEL_DIM)
